# Initial kernel scaffold; baseline (speedup 1.0000x reference)
import functools

import jax
import jax.numpy as jnp
from jax import lax
from jax.experimental import pallas as pl
from jax.experimental.pallas import tpu as pltpu
from jax.experimental.pallas import tpu_sc as plsc

_EMB_DIM = 32
_NUM_CODES = 8192
_ROWS = 8192
_M_BLK = 256
_GRID = _ROWS // _M_BLK
_LOSS_SCALE = 1.25 / (_ROWS * _EMB_DIM)

_NC = 2
_NS = 16
_NW = _NC * _NS
_CHUNK = 128
_CPW = _ROWS // (_NW * _CHUNK)


def _argmin_body(x_ref, embt_ref, idx_ref, loss_ref):
    i = pl.program_id(0)
    x = x_ref[...]
    embt = embt_ref[...]
    xnorm = jnp.sum(x * x, axis=1, keepdims=True)
    enorm = jnp.sum(embt * embt, axis=0, keepdims=True)
    mm = lax.dot_general(x, embt, (((1,), (0,)), ((), ())),
                         preferred_element_type=jnp.float32)
    dist = (xnorm + enorm) - 2.0 * mm
    minval = jnp.min(dist, axis=1, keepdims=True)
    iota = lax.broadcasted_iota(jnp.int32, dist.shape, 1)
    idx = jnp.min(jnp.where(dist == minval, iota, jnp.int32(2**31 - 1)),
                  axis=1, keepdims=True)
    idx_ref[...] = idx

    @pl.when(i == 0)
    def _init():
        loss_ref[0, 0] = 0.0

    loss_ref[0, 0] += jnp.sum(minval) * _LOSS_SCALE


_argmin_call = pl.pallas_call(
    _argmin_body,
    grid=(_GRID,),
    in_specs=[
        pl.BlockSpec((_M_BLK, _EMB_DIM), lambda i: (i, 0)),
        pl.BlockSpec((_EMB_DIM, _NUM_CODES), lambda i: (0, 0)),
    ],
    out_specs=[
        pl.BlockSpec((_M_BLK, 1), lambda i: (i, 0)),
        pl.BlockSpec((1, 1), lambda i: (0, 0)),
    ],
    out_shape=[
        jax.ShapeDtypeStruct((_ROWS, 1), jnp.int32),
        jax.ShapeDtypeStruct((1, 1), jnp.float32),
    ],
)


def _gather_body(emb_hbm, idx_hbm, out_hbm, idx_v, rows_v, sem):
    wid = lax.axis_index("s") * _NC + lax.axis_index("c")
    base = wid * _CPW
    pltpu.sync_copy(idx_hbm.at[pl.ds(base, _CPW)], idx_v)
    copies = [pltpu.async_copy(emb_hbm.at[idx_v.at[j]], rows_v.at[j], sem)
              for j in range(_CPW)]
    for cp in copies:
        cp.wait()
    pltpu.sync_copy(rows_v, out_hbm.at[pl.ds(base, _CPW)])


_gather_call = functools.partial(
    pl.kernel,
    out_type=jax.ShapeDtypeStruct((_NW * _CPW, _CHUNK, _EMB_DIM), jnp.float32),
    mesh=plsc.VectorSubcoreMesh(core_axis_name="c", subcore_axis_name="s"),
    scratch_types=[
        pltpu.VMEM((_CPW, _CHUNK), jnp.int32),
        pltpu.VMEM((_CPW, _CHUNK, _EMB_DIM), jnp.float32),
        pltpu.SemaphoreType.DMA,
    ],
)(_gather_body)


def kernel(x, emb):
    flat_x = x.reshape(_ROWS, _EMB_DIM)
    embt = emb.T
    idx2d, loss11 = _argmin_call(flat_x, embt)
    idx = idx2d.reshape(_NW * _CPW, _CHUNK)
    quant = _gather_call(emb, idx)
    quantized = quant.reshape(x.shape)
    loss = loss11[0, 0]
    return quantized, loss


# speedup vs baseline: 1.1273x; 1.1273x over previous
import functools

import jax
import jax.numpy as jnp
from jax import lax
from jax.experimental import pallas as pl
from jax.experimental.pallas import tpu as pltpu
from jax.experimental.pallas import tpu_sc as plsc

_EMB_DIM = 32
_NUM_CODES = 8192
_ROWS = 8192
_M_BLK = 256
_GRID = _ROWS // _M_BLK
_LOSS_SCALE = 1.25 / (_ROWS * _EMB_DIM)

_NC = 2
_NS = 16
_NW = _NC * _NS
_CHUNK = 128
_CPW = _ROWS // (_NW * _CHUNK)


def _round_to_bf16_f32(v):
    u = lax.bitcast_convert_type(v, jnp.uint32)
    return lax.bitcast_convert_type(u & jnp.uint32(0xFFFF0000), jnp.float32)


def _argmin_body(xt_ref, embt_ref, xnorm_ref, enorm_ref, idx_ref, loss_ref):
    i = pl.program_id(0)
    xt = xt_ref[...]
    embt = embt_ref[...]
    xnorm = xnorm_ref[...]
    enorm = enorm_ref[...]
    mm = lax.dot_general(xt.astype(jnp.bfloat16), embt.astype(jnp.bfloat16),
                         (((0,), (0,)), ((), ())),
                         preferred_element_type=jnp.float32)
    dist = (xnorm + enorm) - 2.0 * mm
    minval = jnp.min(dist, axis=1, keepdims=True)
    iota = lax.broadcasted_iota(jnp.int32, dist.shape, 1)
    idx = jnp.min(jnp.where(dist == minval, iota, jnp.int32(2**31 - 1)),
                  axis=1, keepdims=True)
    idx_ref[...] = idx

    @pl.when(i == 0)
    def _init():
        loss_ref[...] = jnp.zeros_like(loss_ref)

    loss_ref[...] += (jnp.sum(minval) * _LOSS_SCALE).reshape(1, 1)


_argmin_call = pl.pallas_call(
    _argmin_body,
    grid=(_GRID,),
    in_specs=[
        pl.BlockSpec((_EMB_DIM, _M_BLK), lambda i: (0, i)),
        pl.BlockSpec((_EMB_DIM, _NUM_CODES), lambda i: (0, 0)),
        pl.BlockSpec((_M_BLK, 1), lambda i: (i, 0)),
        pl.BlockSpec((1, _NUM_CODES), lambda i: (0, 0)),
    ],
    out_specs=[
        pl.BlockSpec((_M_BLK, 1), lambda i: (i, 0)),
        pl.BlockSpec((1, 1), lambda i: (0, 0)),
    ],
    out_shape=[
        jax.ShapeDtypeStruct((_ROWS, 1), jnp.int32),
        jax.ShapeDtypeStruct((1, 1), jnp.float32),
    ],
)


def _gather_body(emb_hbm, idx_hbm, out_hbm, idx_v, rows_v, sem):
    wid = lax.axis_index("s") * _NC + lax.axis_index("c")
    base = wid * _CPW
    pltpu.sync_copy(idx_hbm.at[pl.ds(base, _CPW)], idx_v)
    copies = [pltpu.async_copy(emb_hbm.at[idx_v.at[j]], rows_v.at[j], sem)
              for j in range(_CPW)]
    for cp in copies:
        cp.wait()
    pltpu.sync_copy(rows_v, out_hbm.at[pl.ds(base, _CPW)])


@functools.cache
def _make_gather_call():
    return functools.partial(
        pl.kernel,
        out_type=jax.ShapeDtypeStruct((_NW * _CPW, _CHUNK, _EMB_DIM),
                                      jnp.float32),
        mesh=plsc.VectorSubcoreMesh(core_axis_name="c", subcore_axis_name="s"),
        scratch_types=[
            pltpu.VMEM((_CPW, _CHUNK), jnp.int32),
            pltpu.VMEM((_CPW, _CHUNK, _EMB_DIM), jnp.float32),
            pltpu.SemaphoreType.DMA,
        ],
        compiler_params=pltpu.CompilerParams(use_tc_tiling_on_sc=False),
    )(_gather_body)


def kernel(x, emb):
    flat_x = x.reshape(_ROWS, _EMB_DIM)
    xnorm = jnp.sum(flat_x ** 2, axis=1, keepdims=True)
    enorm = jnp.sum(emb ** 2, axis=1).reshape(1, _NUM_CODES)
    idx2d, loss11 = _argmin_call(flat_x.T, emb.T, xnorm, enorm)
    idx = idx2d.reshape(_NW * _CPW, _CHUNK)
    quant = _make_gather_call()(emb, idx)
    quantized = quant.reshape(x.shape)
    loss = loss11[0, 0]
    return quantized, loss
